# Initial kernel scaffold; baseline (speedup 1.0000x reference)
import jax
import jax.numpy as jnp
from jax.experimental import pallas as pl


def kernel(history_embedding, candidate_news_representation, user_history_graph, user_history_category_mask, user_history_category_indices, proxy_node_embedding, W_gcn, b_gcn, W_K, W_Q, b_Q, W_aff, b_aff, W_ck, W_cq, b_cq):
    raise NotImplementedError("write your pallas kernel here")


# speedup vs baseline: 5.8486x; 5.8486x over previous
import functools

import jax
import jax.numpy as jnp
from jax.experimental import pallas as pl

B = 256
NN = 5
H = 50
CATN = 18
C = CATN + 1
D = 400
AD = 128
NODES = H + CATN
L = 2
BB = 4

_INV_SCALE = 1.0 / (AD ** 0.5)


def _sue_kernel(hist_ref, cand_ref, graph_ref, maskf_ref, idx_ref,
                proxy_ref, Wg_ref, bg_ref, WK_ref, WQ_ref, bQ_ref,
                Waff_ref, baff_ref, Wck_ref, Wcq_ref, bcq_ref, out_ref):
    proxy = proxy_ref[...]

    g_all = jnp.concatenate(
        [jnp.concatenate([hist_ref[u], proxy], axis=0) for u in range(BB)],
        axis=0)
    h0_all = g_all
    for l in range(L):
        Wl = Wg_ref[l]
        bl = bg_ref[l]
        aggs = [jnp.dot(graph_ref[u], g_all[u * NODES:(u + 1) * NODES],
                        preferred_element_type=jnp.float32)
                for u in range(BB)]
        agg_all = jnp.concatenate(aggs, axis=0)
        g_all = g_all + jax.nn.relu(
            jnp.dot(agg_all, Wl, preferred_element_type=jnp.float32) + bl)
    gfa = g_all + h0_all

    gf_all = jnp.concatenate(
        [gfa[u * NODES:u * NODES + H] for u in range(BB)], axis=0)

    K_all = jnp.dot(gf_all, WK_ref[...],
                    preferred_element_type=jnp.float32)
    cand_all = cand_ref[...].reshape(BB * NN, D)
    Q_all = jnp.dot(cand_all, WQ_ref[...],
                    preferred_element_type=jnp.float32) + bQ_ref[...]

    intra_list = []
    for u in range(BB):
        gf_u = gf_all[u * H:(u + 1) * H]
        K_u = K_all[u * H:(u + 1) * H]
        Q_u = Q_all[u * NN:(u + 1) * NN]
        a = jax.lax.dot_general(
            Q_u, K_u, (((1,), (1,)), ((), ())),
            preferred_element_type=jnp.float32) * _INV_SCALE

        idx_u = idx_ref[u]
        cat_iota = jax.lax.broadcasted_iota(jnp.int32, (C, H), 0)
        onehot = (cat_iota == idx_u).astype(jnp.float32)

        masked = jnp.where(onehot[None, :, :] > 0, a[:, None, :], -1e30)
        M = jnp.max(masked, axis=-1)
        m_h = jnp.dot(M, onehot, preferred_element_type=jnp.float32)
        ex = jnp.exp(a - m_h)
        ssum = jax.lax.dot_general(
            ex, onehot, (((1,), (1,)), ((), ())),
            preferred_element_type=jnp.float32)
        denom = jnp.dot(ssum, onehot,
                        preferred_element_type=jnp.float32) + 1e-12
        alpha = ex / denom

        wfull = (alpha[:, None, :] * onehot[None, :, :]).reshape(NN * C, H)
        intra_list.append(jnp.dot(wfull, gf_u,
                                  preferred_element_type=jnp.float32))

    intra_all = jnp.concatenate(intra_list, axis=0)
    intra2_all = jax.nn.relu(
        jnp.dot(intra_all, Waff_ref[...],
                preferred_element_type=jnp.float32) + baff_ref[...]) + intra_all
    Kc_all = jnp.dot(intra2_all, Wck_ref[...],
                     preferred_element_type=jnp.float32)
    Qc_all = jnp.dot(cand_all, Wcq_ref[...],
                     preferred_element_type=jnp.float32) + bcq_ref[...]

    for u in range(BB):
        Kc_u = Kc_all[u * NN * C:(u + 1) * NN * C].reshape(NN, C, AD)
        Qc_u = Qc_all[u * NN:(u + 1) * NN]
        e = jnp.sum(Kc_u * Qc_u[:, None, :], axis=-1) * _INV_SCALE
        e = jnp.where(maskf_ref[u] > 0, e, -1e9)
        e = e - jnp.max(e, axis=-1, keepdims=True)
        we = jnp.exp(e)
        w = we / jnp.sum(we, axis=-1, keepdims=True)
        intra2_u = intra2_all[u * NN * C:(u + 1) * NN * C].reshape(NN, C, D)
        out_ref[u] = jnp.sum(w[:, :, None] * intra2_u, axis=1)


@functools.partial(jax.jit, static_argnames=("interpret",))
def _sue_pallas(hist, cand, graph, maskf, idx, proxy, W_gcn, b_gcn, W_K,
                W_Q, b_Q, W_aff, b_aff, W_ck, W_cq, b_cq, interpret=False):
    grid = (B // BB,)
    data_spec3 = lambda s1, s2: pl.BlockSpec((BB, s1, s2), lambda i: (i, 0, 0))
    w_spec = lambda shape: pl.BlockSpec(shape, lambda i: (0,) * len(shape))
    return pl.pallas_call(
        _sue_kernel,
        grid=grid,
        in_specs=[
            data_spec3(H, D),
            data_spec3(NN, D),
            data_spec3(NODES, NODES),
            data_spec3(1, C),
            data_spec3(1, H),
            w_spec((CATN, D)),
            w_spec((L, D, D)),
            w_spec((L, D)),
            w_spec((D, AD)),
            w_spec((D, AD)),
            w_spec((1, AD)),
            w_spec((D, D)),
            w_spec((1, D)),
            w_spec((D, AD)),
            w_spec((D, AD)),
            w_spec((1, AD)),
        ],
        out_specs=data_spec3(NN, D),
        out_shape=jax.ShapeDtypeStruct((B, NN, D), jnp.float32),
        interpret=interpret,
    )(hist, cand, graph, maskf, idx, proxy, W_gcn, b_gcn, W_K, W_Q, b_Q,
      W_aff, b_aff, W_ck, W_cq, b_cq)


def kernel(history_embedding, candidate_news_representation, user_history_graph,
           user_history_category_mask, user_history_category_indices,
           proxy_node_embedding, W_gcn, b_gcn, W_K, W_Q, b_Q, W_aff, b_aff,
           W_ck, W_cq, b_cq, interpret=False):
    maskf = user_history_category_mask.at[:, -1].set(1)
    maskf = (maskf > 0).astype(jnp.float32).reshape(B, 1, C)
    idx = user_history_category_indices.astype(jnp.int32).reshape(B, 1, H)
    return _sue_pallas(
        history_embedding, candidate_news_representation, user_history_graph,
        maskf, idx, proxy_node_embedding, W_gcn, b_gcn, W_K, W_Q,
        b_Q.reshape(1, AD), W_aff, b_aff.reshape(1, D), W_ck, W_cq,
        b_cq.reshape(1, AD), interpret=interpret)
